# baseline (speedup 1.0000x reference)
import jax
import jax.numpy as jnp
from jax.experimental import pallas as pl

_BN = 4096


def _body(dt_ref, emb_ref, out_ref):
    row = emb_ref[pl.ds(pl.program_id(0), 1), :]
    emb = jnp.broadcast_to(row, (128, 13))
    for g in range(out_ref.shape[0]):
        for k in range(_BN // 128):
            x = dt_ref[0, :, g, pl.ds(k * 128, 128)]
            y = x.T
            out_ref[g, 0, pl.ds(k * 128, 128), :] = jnp.concatenate(
                [y, emb], axis=-1)


@jax.jit
def kernel(data, emb_table):
    g, n, t, f = data.shape
    e = emb_table.shape[1]
    dt = jnp.transpose(data, (2, 3, 0, 1))
    out_t = pl.pallas_call(
        _body,
        grid=(t, (n + _BN - 1) // _BN),
        in_specs=[
            pl.BlockSpec((1, f, g, _BN), lambda i, j: (i, 0, 0, j)),
            pl.BlockSpec((t, e), lambda i, j: (0, 0)),
        ],
        out_specs=pl.BlockSpec((g, 1, _BN, f + e), lambda i, j: (0, i, j, 0)),
        out_shape=jax.ShapeDtypeStruct((g, t, n, f + e), data.dtype),
    )(dt, emb_table)
    return jnp.transpose(out_t, (0, 2, 1, 3))


# speedup vs baseline: 1.0915x; 1.0915x over previous
import jax
import jax.numpy as jnp
from jax.experimental import pallas as pl

_BN = 2560


def _body(dt_ref, emb_ref, out_ref):
    row = emb_ref[pl.ds(pl.program_id(0), 1), :]
    emb = jnp.broadcast_to(row, (128, 13))
    for g in range(out_ref.shape[0]):
        for k in range(_BN // 128):
            x = dt_ref[0, :, g, pl.ds(k * 128, 128)]
            y = x.T
            out_ref[g, 0, pl.ds(k * 128, 128), :] = jnp.concatenate(
                [y, emb], axis=-1)


@jax.jit
def kernel(data, emb_table):
    g, n, t, f = data.shape
    e = emb_table.shape[1]
    dt = jnp.transpose(data, (2, 3, 0, 1))
    out_t = pl.pallas_call(
        _body,
        grid=(t, (n + _BN - 1) // _BN),
        in_specs=[
            pl.BlockSpec((1, f, g, _BN), lambda i, j: (i, 0, 0, j)),
            pl.BlockSpec((t, e), lambda i, j: (0, 0)),
        ],
        out_specs=pl.BlockSpec((g, 1, _BN, f + e), lambda i, j: (0, i, j, 0)),
        out_shape=jax.ShapeDtypeStruct((g, t, n, f + e), data.dtype),
    )(dt, emb_table)
    return jnp.transpose(out_t, (0, 2, 1, 3))
